# baseline (speedup 1.0000x reference)
import functools

import jax
import jax.numpy as jnp
from jax import lax
from jax.experimental import pallas as pl
from jax.experimental.pallas import tpu as pltpu
from jax.experimental.pallas import tpu_sc as plsc

_B = 16384
_D = 4
_NC = 2
_NS = 16
_NW = _NC * _NS
_BPW = _B // _NW
_EPW = _BPW * _D
_CH = 128
_NCH = _EPW // _CH
_VPC = _CH // 16


@functools.partial(
    pl.kernel,
    out_type=jax.ShapeDtypeStruct((_B * _D,), jnp.float32),
    mesh=plsc.VectorSubcoreMesh(core_axis_name="c", subcore_axis_name="s"),
    compiler_params=pltpu.CompilerParams(needs_layout_passes=False),
    scratch_types=[
        pltpu.VMEM((_BPW,), jnp.int32),
        pltpu.VMEM((_NCH, _CH), jnp.int32),
        pltpu.VMEM((_EPW,), jnp.float32),
        pltpu.SemaphoreType.DMA,
    ],
)
def _sc_gather(idx_hbm, tbl_hbm, out_hbm, idx_v, fidx_v, val_v, sem):
    wid = lax.axis_index("s") * _NC + lax.axis_index("c")
    pltpu.sync_copy(idx_hbm.at[pl.ds(wid * _BPW, _BPW)], idx_v)
    lane = lax.iota(jnp.int32, 16)
    sub = lane >> 2
    off = lane & 3
    copies = []
    for c in range(_NCH):
        for k in range(_VPC):
            i_vec = (c * 32 + k * 4) + sub
            idxv = plsc.load_gather(idx_v, [i_vec])
            fidx_v[c, pl.ds(k * 16, 16)] = idxv * _D + off
        copies.append(
            pltpu.async_copy(
                tbl_hbm.at[fidx_v.at[c]],
                val_v.at[pl.ds(c * _CH, _CH)],
                sem,
            )
        )
    for cp in copies:
        cp.wait()
    pltpu.sync_copy(val_v, out_hbm.at[pl.ds(wid * _EPW, _EPW)])


def kernel(states, tbl):
    idx = jnp.reshape(states.astype(jnp.int32), (_B,))
    tbl_flat = jnp.reshape(tbl, (tbl.size,))
    return jnp.reshape(_sc_gather(idx, tbl_flat), (_B, _D))


# speedup vs baseline: 15.7204x; 15.7204x over previous
import functools

import jax
import jax.numpy as jnp
from jax import lax
from jax.experimental import pallas as pl
from jax.experimental.pallas import tpu as pltpu
from jax.experimental.pallas import tpu_sc as plsc

_B = 16384
_D = 4
_R = 1000000
_RP = 1000192
_NC = 2
_NS = 16
_NW = _NC * _NS
_BPW = _B // _NW
_BLK = 128
_NBLK = _BPW // _BLK


@functools.partial(
    pl.kernel,
    out_type=jax.ShapeDtypeStruct((_B // _BLK * _D, _BLK), jnp.float32),
    mesh=plsc.VectorSubcoreMesh(core_axis_name="c", subcore_axis_name="s"),
    compiler_params=pltpu.CompilerParams(needs_layout_passes=False),
    scratch_types=[
        pltpu.VMEM((_BPW,), jnp.int32),
        pltpu.VMEM((_NBLK * _D, _BLK), jnp.int32),
        pltpu.VMEM((_NBLK * _D, _BLK), jnp.float32),
        pltpu.SemaphoreType.DMA,
    ],
)
def _sc_gather(idx_hbm, tbl_hbm, out_hbm, idx_v, fidx_v, val_v, sem):
    wid = lax.axis_index("s") * _NC + lax.axis_index("c")
    pltpu.sync_copy(idx_hbm.at[pl.ds(wid * _BPW, _BPW)], idx_v)
    copies = []
    for b in range(_NBLK):
        for k in range(_BLK // 16):
            iv = idx_v[pl.ds(b * _BLK + k * 16, 16)]
            base = ((iv >> 7) << 9) + (iv & 127)
            for a in range(_D):
                fidx_v[b * _D + a, pl.ds(k * 16, 16)] = base + (a << 7)
        for a in range(_D):
            copies.append(
                pltpu.async_copy(
                    tbl_hbm.at[fidx_v.at[b * _D + a]],
                    val_v.at[b * _D + a],
                    sem,
                )
            )
    for cp in copies:
        cp.wait()
    pltpu.sync_copy(val_v, out_hbm.at[pl.ds(wid * _NBLK * _D, _NBLK * _D)])


def kernel(states, tbl):
    idx = jnp.reshape(states.astype(jnp.int32), (_B,))
    tbl_p = jnp.pad(tbl, ((0, _RP - _R), (0, 0)))
    tbl_n = jnp.reshape(
        jnp.transpose(jnp.reshape(tbl_p, (_RP // _BLK, _BLK, _D)), (0, 2, 1)),
        (_RP * _D,),
    )
    out_k = _sc_gather(idx, tbl_n)
    return jnp.reshape(
        jnp.transpose(jnp.reshape(out_k, (_B // _BLK, _D, _BLK)), (0, 2, 1)),
        (_B, _D),
    )


# speedup vs baseline: 24.2816x; 1.5446x over previous
import functools

import jax
import jax.numpy as jnp
from jax import lax
from jax.experimental import pallas as pl
from jax.experimental.pallas import tpu as pltpu
from jax.experimental.pallas import tpu_sc as plsc

_B = 16384
_D = 4
_R = 1000000
_NC = 2
_NS = 16
_NW = _NC * _NS
_BPW = _B // _NW
_BLK = 128
_NBLK = _BPW // _BLK


@functools.partial(
    pl.kernel,
    out_type=jax.ShapeDtypeStruct((_B // _BLK * _D, _BLK), jnp.float32),
    mesh=plsc.VectorSubcoreMesh(core_axis_name="c", subcore_axis_name="s"),
    compiler_params=pltpu.CompilerParams(needs_layout_passes=False),
    scratch_types=[
        pltpu.VMEM((_BPW,), jnp.int32),
        pltpu.VMEM((_NBLK * _D, _BLK), jnp.int32),
        pltpu.VMEM((_NBLK * _D, _BLK), jnp.float32),
        pltpu.SemaphoreType.DMA,
    ],
)
def _sc_gather(idx_hbm, tbl_hbm, out_hbm, idx_v, fidx_v, val_v, sem):
    wid = lax.axis_index("s") * _NC + lax.axis_index("c")
    pltpu.sync_copy(idx_hbm.at[pl.ds(wid * _BPW, _BPW)], idx_v)
    copies = []
    for b in range(_NBLK):
        for k in range(_BLK // 16):
            iv = idx_v[pl.ds(b * _BLK + k * 16, 16)]
            for a in range(_D):
                fidx_v[b * _D + a, pl.ds(k * 16, 16)] = iv + (a * _R)
        for a in range(_D):
            copies.append(
                pltpu.async_copy(
                    tbl_hbm.at[fidx_v.at[b * _D + a]],
                    val_v.at[b * _D + a],
                    sem,
                )
            )
    for cp in copies:
        cp.wait()
    pltpu.sync_copy(val_v, out_hbm.at[pl.ds(wid * _NBLK * _D, _NBLK * _D)])


def kernel(states, tbl):
    idx = jnp.reshape(states.astype(jnp.int32), (_B,))
    tbl_n = jnp.reshape(jnp.transpose(tbl), (_R * _D,))
    out_k = _sc_gather(idx, tbl_n)
    return jnp.reshape(
        jnp.transpose(jnp.reshape(out_k, (_B // _BLK, _D, _BLK)), (0, 2, 1)),
        (_B, _D),
    )
